# baseline (speedup 1.0000x reference)
import jax
import jax.numpy as jnp
from jax import lax
from jax.experimental import pallas as pl
from jax.experimental.pallas import tpu as pltpu
from jax.experimental.pallas import tpu_sc as plsc

BATCH = 4096
FAST_DIM = 64
VOCAB = 100000

_NC = 2
_NS = 16
_NW = _NC * _NS
_B_PER_W = BATCH // _NW

_BN = 512
_NFULL = VOCAB // _BN
_TAIL = VOCAB - _NFULL * _BN
_NBUF = 4


def _sc_gather_body(idx_hbm, table_hbm, out_hbm, idx_v, rows_v, sem):
    wid = lax.axis_index("s") * _NC + lax.axis_index("c")
    base = wid * _B_PER_W
    pltpu.sync_copy(idx_hbm.at[pl.ds(base, _B_PER_W)], idx_v)
    pltpu.async_copy(table_hbm.at[idx_v], rows_v, sem).wait()
    pltpu.sync_copy(rows_v, out_hbm.at[pl.ds(base, _B_PER_W)])


def _sc_gather(pair_idx, table2):
    mesh = plsc.VectorSubcoreMesh(core_axis_name="c", subcore_axis_name="s")
    return pl.kernel(
        _sc_gather_body,
        mesh=mesh,
        out_type=jax.ShapeDtypeStruct((BATCH, 2 * FAST_DIM), jnp.float32),
        scratch_types=[
            pltpu.VMEM((_B_PER_W,), jnp.int32),
            pltpu.VMEM((_B_PER_W, 2 * FAST_DIM), jnp.float32),
            pltpu.SemaphoreType.DMA,
        ],
    )(pair_idx, table2)


def _full_copy(acc, out_hbm, sems, slot, step):
    return pltpu.make_async_copy(
        acc.at[slot],
        out_hbm.at[:, pl.ds(step * _BN, _BN)],
        sems.at[slot],
    )


def _select_half(packed, parity):
    lo = packed[:, :FAST_DIM]
    hi = packed[:, FAST_DIM:]
    return lo + parity * (hi - lo)


def _mm_body(packed_ref, par_ref, w_ref, b_ref, out_hbm, emb_vmem, acc, sems):
    i = pl.program_id(0)
    slot = lax.rem(i, _NBUF)

    @pl.when(i == 0)
    def _():
        emb_vmem[...] = _select_half(packed_ref[...], par_ref[...])

    @pl.when(i >= _NBUF)
    def _():
        _full_copy(acc, out_hbm, sems, slot, i - _NBUF).wait()

    acc[slot] = (
        jnp.dot(emb_vmem[...], w_ref[...], preferred_element_type=jnp.float32)
        + b_ref[...]
    )

    _full_copy(acc, out_hbm, sems, slot, i).start()

    @pl.when(i == _NFULL - 1)
    def _():
        for k in range(_NBUF):
            s = _NFULL - _NBUF + k
            _full_copy(acc, out_hbm, sems, s % _NBUF, s).wait()


def _tc_project(packed, parity, W, b2d):
    return pl.pallas_call(
        _mm_body,
        grid=(_NFULL,),
        in_specs=[
            pl.BlockSpec((BATCH, 2 * FAST_DIM), lambda i: (0, 0)),
            pl.BlockSpec((BATCH, 1), lambda i: (0, 0)),
            pl.BlockSpec((FAST_DIM, _BN), lambda i: (0, i)),
            pl.BlockSpec((1, _BN), lambda i: (0, i)),
        ],
        out_specs=pl.BlockSpec(memory_space=pltpu.HBM),
        out_shape=jax.ShapeDtypeStruct((BATCH, VOCAB), jnp.float32),
        scratch_shapes=[
            pltpu.VMEM((BATCH, FAST_DIM), jnp.float32),
            pltpu.VMEM((_NBUF, BATCH, _BN), jnp.float32),
            pltpu.SemaphoreType.DMA((_NBUF,)),
        ],
        compiler_params=pltpu.CompilerParams(
            dimension_semantics=("arbitrary",),
        ),
    )(packed, parity, W, b2d)


def _tail_body(logits_ref, packed_ref, par_ref, w_ref, b_ref, out_ref):
    del logits_ref
    emb = _select_half(packed_ref[...], par_ref[...])
    out_ref[...] = (
        jnp.dot(emb, w_ref[...], preferred_element_type=jnp.float32)
        + b_ref[...]
    )


def _tc_tail(logits, packed, parity, W, b2d):
    blk = 256
    last = VOCAB // blk
    return pl.pallas_call(
        _tail_body,
        grid=(1,),
        in_specs=[
            pl.BlockSpec(memory_space=pltpu.HBM),
            pl.BlockSpec((BATCH, 2 * FAST_DIM), lambda i: (0, 0)),
            pl.BlockSpec((BATCH, 1), lambda i: (0, 0)),
            pl.BlockSpec((FAST_DIM, blk), lambda i: (0, last)),
            pl.BlockSpec((1, blk), lambda i: (0, last)),
        ],
        out_specs=pl.BlockSpec((BATCH, blk), lambda i: (0, last)),
        out_shape=jax.ShapeDtypeStruct((BATCH, VOCAB), jnp.float32),
        input_output_aliases={0: 0},
        compiler_params=pltpu.CompilerParams(
            dimension_semantics=("arbitrary",),
        ),
    )(logits, packed, parity, W, b2d)


def kernel(x, emb_table, W, b):
    last_tok = x[:, -1].astype(jnp.int32)
    pair_idx = last_tok // 2
    parity = (last_tok & 1).astype(jnp.float32).reshape(BATCH, 1)
    table2 = emb_table.reshape(VOCAB // 2, 2 * FAST_DIM)
    packed = _sc_gather(pair_idx, table2)
    b2d = b.reshape(1, VOCAB)
    logits = _tc_project(packed, parity, W, b2d)
    return _tc_tail(logits, packed, parity, W, b2d)


# speedup vs baseline: 3.0869x; 3.0869x over previous
import jax
import jax.numpy as jnp
from jax import lax
from jax.experimental import pallas as pl
from jax.experimental.pallas import tpu as pltpu
from jax.experimental.pallas import tpu_sc as plsc

BATCH = 4096
FAST_DIM = 64
VOCAB = 100000

_NC = 2
_NS = 16
_NW = _NC * _NS
_B_PER_W = BATCH // _NW

_BN = 512
_NSTEPS = pl.cdiv(VOCAB, _BN)
_TAIL = VOCAB - (_NSTEPS - 1) * _BN
_NBUF = 3


def _sc_gather_body(idx_hbm, table_hbm, out_hbm, idx_v, rows_v, sem):
    wid = lax.axis_index("s") * _NC + lax.axis_index("c")
    base = wid * _B_PER_W
    pltpu.sync_copy(idx_hbm.at[pl.ds(base, _B_PER_W)], idx_v)
    pltpu.async_copy(table_hbm.at[idx_v], rows_v, sem).wait()
    pltpu.sync_copy(rows_v, out_hbm.at[pl.ds(base, _B_PER_W)])


def _sc_gather(pair_idx, table2):
    mesh = plsc.VectorSubcoreMesh(core_axis_name="c", subcore_axis_name="s")
    return pl.kernel(
        _sc_gather_body,
        mesh=mesh,
        out_type=jax.ShapeDtypeStruct((BATCH, 2 * FAST_DIM), jnp.float32),
        scratch_types=[
            pltpu.VMEM((_B_PER_W,), jnp.int32),
            pltpu.VMEM((_B_PER_W, 2 * FAST_DIM), jnp.float32),
            pltpu.SemaphoreType.DMA,
        ],
    )(pair_idx, table2)


def _slab_copy(acc, out_hbm, sems, slot, step):
    return pltpu.make_async_copy(
        acc.at[slot],
        out_hbm.at[pl.ds(step * _BN, _BN)],
        sems.at[slot],
    )


def _tail_copy(acc, out_hbm, sems, slot):
    return pltpu.make_async_copy(
        acc.at[slot, pl.ds(0, _TAIL)],
        out_hbm.at[pl.ds((_NSTEPS - 1) * _BN, _TAIL)],
        sems.at[slot],
    )


def _select_half(packed, parity):
    lo = packed[:, :FAST_DIM]
    hi = packed[:, FAST_DIM:]
    return lo + parity * (hi - lo)


def _mm_body(packed_ref, par_ref, w_ref, b_ref, out_hbm, embt_vmem, acc, sems):
    i = pl.program_id(0)
    slot = lax.rem(i, _NBUF)

    @pl.when(i == 0)
    def _():
        embt_vmem[...] = _select_half(packed_ref[...], par_ref[...]).T

    @pl.when(i >= _NBUF)
    def _():
        _slab_copy(acc, out_hbm, sems, slot, i - _NBUF).wait()

    acc[slot] = (
        lax.dot_general(
            w_ref[...], embt_vmem[...],
            (((0,), (0,)), ((), ())),
            preferred_element_type=jnp.float32,
        )
        + b_ref[...]
    )

    @pl.when(i < _NSTEPS - 1)
    def _():
        _slab_copy(acc, out_hbm, sems, slot, i).start()

    @pl.when(i == _NSTEPS - 1)
    def _():
        _tail_copy(acc, out_hbm, sems, slot).start()
        for k in range(_NBUF):
            s = _NSTEPS - _NBUF + k
            if s == _NSTEPS - 1:
                _tail_copy(acc, out_hbm, sems, s % _NBUF).wait()
            else:
                _slab_copy(acc, out_hbm, sems, s % _NBUF, s).wait()


def _tc_project_t(packed, parity, W, bcol):
    return pl.pallas_call(
        _mm_body,
        grid=(_NSTEPS,),
        in_specs=[
            pl.BlockSpec((BATCH, 2 * FAST_DIM), lambda i: (0, 0)),
            pl.BlockSpec((BATCH, 1), lambda i: (0, 0)),
            pl.BlockSpec((FAST_DIM, _BN), lambda i: (0, i)),
            pl.BlockSpec((_BN, 1), lambda i: (i, 0)),
        ],
        out_specs=pl.BlockSpec(memory_space=pltpu.HBM),
        out_shape=jax.ShapeDtypeStruct((VOCAB, BATCH), jnp.float32),
        scratch_shapes=[
            pltpu.VMEM((FAST_DIM, BATCH), jnp.float32),
            pltpu.VMEM((_NBUF, _BN, BATCH), jnp.float32),
            pltpu.SemaphoreType.DMA((_NBUF,)),
        ],
        compiler_params=pltpu.CompilerParams(
            dimension_semantics=("arbitrary",),
        ),
    )(packed, parity, W, bcol)


def kernel(x, emb_table, W, b):
    last_tok = x[:, -1].astype(jnp.int32)
    pair_idx = last_tok // 2
    parity = (last_tok & 1).astype(jnp.float32).reshape(BATCH, 1)
    table2 = emb_table.reshape(VOCAB // 2, 2 * FAST_DIM)
    packed = _sc_gather(pair_idx, table2)
    logits_t = _tc_project_t(packed, parity, W, b.reshape(VOCAB, 1))
    return logits_t.T


# speedup vs baseline: 3.4269x; 1.1102x over previous
import jax
import jax.numpy as jnp
from jax import lax
from jax.experimental import pallas as pl
from jax.experimental.pallas import tpu as pltpu
from jax.experimental.pallas import tpu_sc as plsc

BATCH = 4096
FAST_DIM = 64
VOCAB = 100000

_NC = 2
_NS = 16
_NW = _NC * _NS
_B_PER_W = BATCH // _NW

_BN = 512
_NSTEPS = pl.cdiv(VOCAB, _BN)
_TAIL = VOCAB - (_NSTEPS - 1) * _BN
_NBUF = 3


def _sc_gather_body(idx_hbm, table_hbm, out_hbm, idx_v, rows_v, sem):
    wid = lax.axis_index("s") * _NC + lax.axis_index("c")
    base = wid * _B_PER_W
    pltpu.sync_copy(idx_hbm.at[pl.ds(base, _B_PER_W)], idx_v)
    pltpu.async_copy(table_hbm.at[idx_v], rows_v, sem).wait()
    pltpu.sync_copy(rows_v, out_hbm.at[pl.ds(base, _B_PER_W)])


def _sc_gather(last_tok, emb_table):
    mesh = plsc.VectorSubcoreMesh(core_axis_name="c", subcore_axis_name="s")
    return pl.kernel(
        _sc_gather_body,
        mesh=mesh,
        out_type=jax.ShapeDtypeStruct((BATCH, FAST_DIM), jnp.float32),
        scratch_types=[
            pltpu.VMEM((_B_PER_W,), jnp.int32),
            pltpu.VMEM((_B_PER_W, FAST_DIM), jnp.float32),
            pltpu.SemaphoreType.DMA,
        ],
        compiler_params=pltpu.CompilerParams(use_tc_tiling_on_sc=False),
    )(last_tok, emb_table)


def _slab_copy(acc, out_hbm, sems, slot, step):
    return pltpu.make_async_copy(
        acc.at[slot],
        out_hbm.at[pl.ds(step * _BN, _BN)],
        sems.at[slot],
    )


def _tail_copy(acc, out_hbm, sems, slot):
    return pltpu.make_async_copy(
        acc.at[slot, pl.ds(0, _TAIL)],
        out_hbm.at[pl.ds((_NSTEPS - 1) * _BN, _TAIL)],
        sems.at[slot],
    )


def _mm_body(emb_ref, w_ref, b_ref, out_hbm, embt_vmem, acc, sems):
    i = pl.program_id(0)
    slot = lax.rem(i, _NBUF)

    @pl.when(i == 0)
    def _():
        embt_vmem[...] = emb_ref[...].T

    @pl.when(i >= _NBUF)
    def _():
        _slab_copy(acc, out_hbm, sems, slot, i - _NBUF).wait()

    acc[slot] = (
        lax.dot_general(
            w_ref[...], embt_vmem[...],
            (((0,), (0,)), ((), ())),
            preferred_element_type=jnp.float32,
        )
        + b_ref[...].T
    )

    @pl.when(i < _NSTEPS - 1)
    def _():
        _slab_copy(acc, out_hbm, sems, slot, i).start()

    @pl.when(i == _NSTEPS - 1)
    def _():
        _tail_copy(acc, out_hbm, sems, slot).start()
        for k in range(_NBUF):
            s = _NSTEPS - _NBUF + k
            if s == _NSTEPS - 1:
                _tail_copy(acc, out_hbm, sems, s % _NBUF).wait()
            else:
                _slab_copy(acc, out_hbm, sems, s % _NBUF, s).wait()


def _tc_project_t(fast_embed, W, brow):
    return pl.pallas_call(
        _mm_body,
        grid=(_NSTEPS,),
        in_specs=[
            pl.BlockSpec((BATCH, FAST_DIM), lambda i: (0, 0)),
            pl.BlockSpec((FAST_DIM, _BN), lambda i: (0, i)),
            pl.BlockSpec((1, _BN), lambda i: (0, i)),
        ],
        out_specs=pl.BlockSpec(memory_space=pltpu.HBM),
        out_shape=jax.ShapeDtypeStruct((VOCAB, BATCH), jnp.float32),
        scratch_shapes=[
            pltpu.VMEM((FAST_DIM, BATCH), jnp.float32),
            pltpu.VMEM((_NBUF, _BN, BATCH), jnp.float32),
            pltpu.SemaphoreType.DMA((_NBUF,)),
        ],
        compiler_params=pltpu.CompilerParams(
            dimension_semantics=("arbitrary",),
        ),
    )(fast_embed, W, brow)


def kernel(x, emb_table, W, b):
    last_tok = x[:, -1].astype(jnp.int32)
    fast_embed = _sc_gather(last_tok, emb_table)
    logits_t = _tc_project_t(fast_embed, W, b.reshape(1, VOCAB))
    return logits_t.T


# speedup vs baseline: 3.7479x; 1.0936x over previous
import jax
import jax.numpy as jnp
from jax import lax
from jax.experimental import pallas as pl
from jax.experimental.pallas import tpu as pltpu
from jax.experimental.pallas import tpu_sc as plsc

BATCH = 4096
FAST_DIM = 64
VOCAB = 100000

_NC = 2
_NS = 16
_NW = _NC * _NS
_D_PER_W = FAST_DIM // _NW
_LANES = 16

_BN = 512
_NSTEPS = pl.cdiv(VOCAB, _BN)
_TAIL = VOCAB - (_NSTEPS - 1) * _BN
_NBUF = 3


def _sc_colgather_body(idx_hbm, tablet_hbm, out_hbm, idx_v, row_v, gath_v, sem):
    wid = lax.axis_index("s") * _NC + lax.axis_index("c")
    pltpu.sync_copy(idx_hbm, idx_v)
    for r in range(_D_PER_W):
        d = wid * _D_PER_W + r
        pltpu.async_copy(tablet_hbm.at[d], row_v, sem).wait()

        def body(j, _):
            iv = idx_v[pl.ds(j * _LANES, _LANES)]
            gath_v[pl.ds(j * _LANES, _LANES)] = plsc.load_gather(row_v, [iv])
            return 0

        lax.fori_loop(0, BATCH // _LANES, body, 0)
        pltpu.sync_copy(gath_v, out_hbm.at[d])


def _sc_gather_t(last_tok, tablet):
    mesh = plsc.VectorSubcoreMesh(core_axis_name="c", subcore_axis_name="s")
    return pl.kernel(
        _sc_colgather_body,
        mesh=mesh,
        out_type=jax.ShapeDtypeStruct((FAST_DIM, BATCH), jnp.float32),
        scratch_types=[
            pltpu.VMEM((BATCH,), jnp.int32),
            pltpu.VMEM((VOCAB,), jnp.float32),
            pltpu.VMEM((BATCH,), jnp.float32),
            pltpu.SemaphoreType.DMA,
        ],
        compiler_params=pltpu.CompilerParams(needs_layout_passes=False),
    )(last_tok, tablet)


def _slab_copy(acc, out_hbm, sems, slot, step):
    return pltpu.make_async_copy(
        acc.at[slot],
        out_hbm.at[pl.ds(step * _BN, _BN)],
        sems.at[slot],
    )


def _tail_copy(acc, out_hbm, sems, slot):
    return pltpu.make_async_copy(
        acc.at[slot, pl.ds(0, _TAIL)],
        out_hbm.at[pl.ds((_NSTEPS - 1) * _BN, _TAIL)],
        sems.at[slot],
    )


def _mm_body(embt_ref, w_ref, b_ref, out_hbm, acc, sems):
    i = pl.program_id(0)
    slot = lax.rem(i, _NBUF)

    @pl.when(i >= _NBUF)
    def _():
        _slab_copy(acc, out_hbm, sems, slot, i - _NBUF).wait()

    acc[slot] = (
        lax.dot_general(
            w_ref[...], embt_ref[...],
            (((0,), (0,)), ((), ())),
            preferred_element_type=jnp.float32,
        )
        + b_ref[...].T
    )

    @pl.when(i < _NSTEPS - 1)
    def _():
        _slab_copy(acc, out_hbm, sems, slot, i).start()

    @pl.when(i == _NSTEPS - 1)
    def _():
        _tail_copy(acc, out_hbm, sems, slot).start()
        for k in range(_NBUF):
            s = _NSTEPS - _NBUF + k
            if s == _NSTEPS - 1:
                _tail_copy(acc, out_hbm, sems, s % _NBUF).wait()
            else:
                _slab_copy(acc, out_hbm, sems, s % _NBUF, s).wait()


def _tc_project_t(embt, W, brow):
    return pl.pallas_call(
        _mm_body,
        grid=(_NSTEPS,),
        in_specs=[
            pl.BlockSpec((FAST_DIM, BATCH), lambda i: (0, 0)),
            pl.BlockSpec((FAST_DIM, _BN), lambda i: (0, i)),
            pl.BlockSpec((1, _BN), lambda i: (0, i)),
        ],
        out_specs=pl.BlockSpec(memory_space=pltpu.HBM),
        out_shape=jax.ShapeDtypeStruct((VOCAB, BATCH), jnp.float32),
        scratch_shapes=[
            pltpu.VMEM((_NBUF, _BN, BATCH), jnp.float32),
            pltpu.SemaphoreType.DMA((_NBUF,)),
        ],
        compiler_params=pltpu.CompilerParams(
            dimension_semantics=("arbitrary",),
        ),
    )(embt, W, brow)


def kernel(x, emb_table, W, b):
    last_tok = x[:, -1].astype(jnp.int32)
    embt = _sc_gather_t(last_tok, emb_table.T)
    logits_t = _tc_project_t(embt, W, b.reshape(1, VOCAB))
    return logits_t.T


# speedup vs baseline: 3.7488x; 1.0003x over previous
import jax
import jax.numpy as jnp
from jax import lax
from jax.experimental import pallas as pl
from jax.experimental.pallas import tpu as pltpu
from jax.experimental.pallas import tpu_sc as plsc

BATCH = 4096
FAST_DIM = 64
VOCAB = 100000

_NC = 2
_NS = 16
_NW = _NC * _NS
_D_PER_W = FAST_DIM // _NW
_LANES = 16

_BN = 512
_NSTEPS = pl.cdiv(VOCAB, _BN)
_TAIL = VOCAB - (_NSTEPS - 1) * _BN
_NBUF = 4


def _sc_colgather_body(idx_hbm, tablet_hbm, out_hbm, idx_v, row_v, gath_v, sem):
    wid = lax.axis_index("s") * _NC + lax.axis_index("c")
    pltpu.sync_copy(idx_hbm, idx_v)
    for r in range(_D_PER_W):
        d = wid * _D_PER_W + r
        pltpu.async_copy(tablet_hbm.at[d], row_v, sem).wait()

        def body(j, _):
            iv = idx_v[pl.ds(j * _LANES, _LANES)]
            gath_v[pl.ds(j * _LANES, _LANES)] = plsc.load_gather(row_v, [iv])
            return 0

        lax.fori_loop(0, BATCH // _LANES, body, 0)
        pltpu.sync_copy(gath_v, out_hbm.at[d])


def _sc_gather_t(last_tok, tablet):
    mesh = plsc.VectorSubcoreMesh(core_axis_name="c", subcore_axis_name="s")
    return pl.kernel(
        _sc_colgather_body,
        mesh=mesh,
        out_type=jax.ShapeDtypeStruct((FAST_DIM, BATCH), jnp.float32),
        scratch_types=[
            pltpu.VMEM((BATCH,), jnp.int32),
            pltpu.VMEM((VOCAB,), jnp.float32),
            pltpu.VMEM((BATCH,), jnp.float32),
            pltpu.SemaphoreType.DMA,
        ],
        compiler_params=pltpu.CompilerParams(needs_layout_passes=False),
    )(last_tok, tablet)


def _slab_copy(acc, out_hbm, sems, slot, step):
    return pltpu.make_async_copy(
        acc.at[slot],
        out_hbm.at[pl.ds(step * _BN, _BN)],
        sems.at[slot],
    )


def _tail_copy(acc, out_hbm, sems, slot):
    return pltpu.make_async_copy(
        acc.at[slot, pl.ds(0, _TAIL)],
        out_hbm.at[pl.ds((_NSTEPS - 1) * _BN, _TAIL)],
        sems.at[slot],
    )


def _mm_body(embt_ref, w_ref, b_ref, out_hbm, acc, sems):
    i = pl.program_id(0)
    slot = lax.rem(i, _NBUF)

    @pl.when(i >= _NBUF)
    def _():
        _slab_copy(acc, out_hbm, sems, slot, i - _NBUF).wait()

    acc[slot] = (
        lax.dot_general(
            w_ref[...], embt_ref[...],
            (((0,), (0,)), ((), ())),
            preferred_element_type=jnp.float32,
        )
        + b_ref[...].T
    )

    @pl.when(i < _NSTEPS - 1)
    def _():
        _slab_copy(acc, out_hbm, sems, slot, i).start()

    @pl.when(i == _NSTEPS - 1)
    def _():
        _tail_copy(acc, out_hbm, sems, slot).start()
        for k in range(_NBUF):
            s = _NSTEPS - _NBUF + k
            if s == _NSTEPS - 1:
                _tail_copy(acc, out_hbm, sems, s % _NBUF).wait()
            else:
                _slab_copy(acc, out_hbm, sems, s % _NBUF, s).wait()


def _tc_project_t(embt, W, brow):
    return pl.pallas_call(
        _mm_body,
        grid=(_NSTEPS,),
        in_specs=[
            pl.BlockSpec((FAST_DIM, BATCH), lambda i: (0, 0)),
            pl.BlockSpec((FAST_DIM, _BN), lambda i: (0, i)),
            pl.BlockSpec((1, _BN), lambda i: (0, i)),
        ],
        out_specs=pl.BlockSpec(memory_space=pltpu.HBM),
        out_shape=jax.ShapeDtypeStruct((VOCAB, BATCH), jnp.float32),
        scratch_shapes=[
            pltpu.VMEM((_NBUF, _BN, BATCH), jnp.float32),
            pltpu.SemaphoreType.DMA((_NBUF,)),
        ],
        compiler_params=pltpu.CompilerParams(
            dimension_semantics=("arbitrary",),
        ),
    )(embt, W, brow)


def kernel(x, emb_table, W, b):
    last_tok = x[:, -1].astype(jnp.int32)
    embt = _sc_gather_t(last_tok, emb_table.T)
    logits_t = _tc_project_t(embt, W, b.reshape(1, VOCAB))
    return logits_t.T


# speedup vs baseline: 3.7615x; 1.0034x over previous
import jax
import jax.numpy as jnp
from jax import lax
from jax.experimental import pallas as pl
from jax.experimental.pallas import tpu as pltpu
from jax.experimental.pallas import tpu_sc as plsc

BATCH = 4096
FAST_DIM = 64
VOCAB = 100000

_NC = 2
_NS = 16
_NW = _NC * _NS
_D_PER_W = FAST_DIM // _NW
_LANES = 16

_BN = 1024
_NSTEPS = pl.cdiv(VOCAB, _BN)
_TAIL = VOCAB - (_NSTEPS - 1) * _BN
_NBUF = 2


def _sc_colgather_body(idx_hbm, tablet_hbm, out_hbm, idx_v, row_v, gath_v, sem):
    wid = lax.axis_index("s") * _NC + lax.axis_index("c")
    pltpu.sync_copy(idx_hbm, idx_v)
    for r in range(_D_PER_W):
        d = wid * _D_PER_W + r
        pltpu.async_copy(tablet_hbm.at[d], row_v, sem).wait()

        def body(j, _):
            iv = idx_v[pl.ds(j * _LANES, _LANES)]
            gath_v[pl.ds(j * _LANES, _LANES)] = plsc.load_gather(row_v, [iv])
            return 0

        lax.fori_loop(0, BATCH // _LANES, body, 0)
        pltpu.sync_copy(gath_v, out_hbm.at[d])


def _sc_gather_t(last_tok, tablet):
    mesh = plsc.VectorSubcoreMesh(core_axis_name="c", subcore_axis_name="s")
    return pl.kernel(
        _sc_colgather_body,
        mesh=mesh,
        out_type=jax.ShapeDtypeStruct((FAST_DIM, BATCH), jnp.float32),
        scratch_types=[
            pltpu.VMEM((BATCH,), jnp.int32),
            pltpu.VMEM((VOCAB,), jnp.float32),
            pltpu.VMEM((BATCH,), jnp.float32),
            pltpu.SemaphoreType.DMA,
        ],
        compiler_params=pltpu.CompilerParams(needs_layout_passes=False),
    )(last_tok, tablet)


def _slab_copy(acc, out_hbm, sems, slot, step):
    return pltpu.make_async_copy(
        acc.at[slot],
        out_hbm.at[pl.ds(step * _BN, _BN)],
        sems.at[slot],
    )


def _tail_copy(acc, out_hbm, sems, slot):
    return pltpu.make_async_copy(
        acc.at[slot, pl.ds(0, _TAIL)],
        out_hbm.at[pl.ds((_NSTEPS - 1) * _BN, _TAIL)],
        sems.at[slot],
    )


def _mm_body(embt_ref, w_ref, b_ref, out_hbm, acc, sems):
    i = pl.program_id(0)
    slot = lax.rem(i, _NBUF)

    @pl.when(i >= _NBUF)
    def _():
        _slab_copy(acc, out_hbm, sems, slot, i - _NBUF).wait()

    acc[slot] = (
        lax.dot_general(
            w_ref[...], embt_ref[...],
            (((0,), (0,)), ((), ())),
            preferred_element_type=jnp.float32,
        )
        + b_ref[...].T
    )

    @pl.when(i < _NSTEPS - 1)
    def _():
        _slab_copy(acc, out_hbm, sems, slot, i).start()

    @pl.when(i == _NSTEPS - 1)
    def _():
        _tail_copy(acc, out_hbm, sems, slot).start()
        for k in range(_NBUF):
            s = _NSTEPS - _NBUF + k
            if s == _NSTEPS - 1:
                _tail_copy(acc, out_hbm, sems, s % _NBUF).wait()
            else:
                _slab_copy(acc, out_hbm, sems, s % _NBUF, s).wait()


def _tc_project_t(embt, W, brow):
    return pl.pallas_call(
        _mm_body,
        grid=(_NSTEPS,),
        in_specs=[
            pl.BlockSpec((FAST_DIM, BATCH), lambda i: (0, 0)),
            pl.BlockSpec((FAST_DIM, _BN), lambda i: (0, i)),
            pl.BlockSpec((1, _BN), lambda i: (0, i)),
        ],
        out_specs=pl.BlockSpec(memory_space=pltpu.HBM),
        out_shape=jax.ShapeDtypeStruct((VOCAB, BATCH), jnp.float32),
        scratch_shapes=[
            pltpu.VMEM((_NBUF, _BN, BATCH), jnp.float32),
            pltpu.SemaphoreType.DMA((_NBUF,)),
        ],
        compiler_params=pltpu.CompilerParams(
            dimension_semantics=("arbitrary",),
        ),
    )(embt, W, brow)


def kernel(x, emb_table, W, b):
    last_tok = x[:, -1].astype(jnp.int32)
    embt = _sc_gather_t(last_tok, emb_table.T)
    logits_t = _tc_project_t(embt, W, b.reshape(1, VOCAB))
    return logits_t.T
